# Initial kernel scaffold; baseline (speedup 1.0000x reference)
import jax
import jax.numpy as jnp
from jax.experimental import pallas as pl


def kernel(x, edge_index, batch, W1, b1, W2, b2, W3, b3, lin1_W, lin1_b, lin2_W, lin2_b):
    raise NotImplementedError("write your pallas kernel here")


# speedup vs baseline: 9.7084x; 9.7084x over previous
import functools

import jax
import jax.numpy as jnp
from jax import lax
from jax.experimental import pallas as pl
from jax.experimental.pallas import tpu as pltpu
from jax.experimental.pallas import tpu_sc as plsc

N = 10000
E = 320000
D = 128
G = 64
OUT = 10

NUM_CORES = 2
NUM_TILES = 16
CHUNK = 128
E_PAD = 323584
DUMMY = N
N_ACC = 10112
ROWS_PER_TILE = N_ACC // NUM_TILES
EDGES_PER_TILE = E_PAD // (NUM_CORES * NUM_TILES)
CHUNKS_PER_TILE = EDGES_PER_TILE // CHUNK

_MESH = plsc.VectorSubcoreMesh(core_axis_name="c", subcore_axis_name="s")


HIST = 10240
HIST_SLICE = HIST // NUM_TILES


def _deg_body(dst_hbm, zeros_hbm, out_hbm,
              dst_v, hist_v, part_v, out_v, cmb_sh, sem):
    c = lax.axis_index("c")
    s = lax.axis_index("s")
    pltpu.sync_copy(zeros_hbm, hist_v)
    base = c * (E_PAD // NUM_CORES) + s * EDGES_PER_TILE
    pltpu.sync_copy(dst_hbm.at[pl.ds(base, EDGES_PER_TILE)], dst_v)
    ones16 = jnp.full((16,), 1.0, jnp.float32)

    def step(j, carry):
        idx = dst_v[pl.ds(j * 16, 16)]
        plsc.addupdate_scatter(hist_v, [idx], ones16)
        return carry

    lax.fori_loop(0, EDGES_PER_TILE // 16, step, 0)
    pltpu.sync_copy(hist_v, cmb_sh.at[pl.ds(s * HIST, HIST)])
    plsc.subcore_barrier()

    for k in range(NUM_TILES):
        pltpu.sync_copy(cmb_sh.at[pl.ds(k * HIST + s * HIST_SLICE, HIST_SLICE)],
                        part_v.at[pl.ds(k * HIST_SLICE, HIST_SLICE)])

    def combine(j, carry):
        v = part_v[pl.ds(j * 16, 16)]
        for k in range(1, NUM_TILES):
            v = v + part_v[pl.ds(k * HIST_SLICE + j * 16, 16)]
        out_v[pl.ds(j * 16, 16)] = v
        return carry

    lax.fori_loop(0, HIST_SLICE // 16, combine, 0)
    pltpu.sync_copy(out_v,
                    out_hbm.at[pl.ds(c * HIST + s * HIST_SLICE, HIST_SLICE)])


_deg_call = pl.kernel(
    _deg_body,
    out_type=jax.ShapeDtypeStruct((2 * HIST,), jnp.float32),
    mesh=_MESH,
    scratch_types=[
        pltpu.VMEM((EDGES_PER_TILE,), jnp.int32),
        pltpu.VMEM((HIST,), jnp.float32),
        pltpu.VMEM((HIST,), jnp.float32),
        pltpu.VMEM((HIST_SLICE,), jnp.float32),
        pltpu.VMEM_SHARED((NUM_TILES * HIST,), jnp.float32),
        pltpu.SemaphoreType.DMA,
    ],
    compiler_params=pltpu.CompilerParams(needs_layout_passes=False),
)


def _msg_body(h_hbm, src_hbm, dst_hbm, zeros_hbm, out_hbm,
              sidx_v, didx_v, rows_v, acc_sh, sem):
    c = lax.axis_index("c")
    s = lax.axis_index("s")
    row0 = s * ROWS_PER_TILE
    pltpu.sync_copy(zeros_hbm.at[pl.ds(row0, ROWS_PER_TILE), :],
                    acc_sh.at[pl.ds(row0, ROWS_PER_TILE), :])
    plsc.subcore_barrier()

    base = c * (E_PAD // NUM_CORES) + s * EDGES_PER_TILE

    def step(i, carry):
        e0 = base + i * CHUNK
        pltpu.sync_copy(src_hbm.at[pl.ds(e0, CHUNK)], sidx_v)
        pltpu.sync_copy(dst_hbm.at[pl.ds(e0, CHUNK)], didx_v)
        pltpu.async_copy(h_hbm.at[sidx_v], rows_v, sem).wait()
        pltpu.sync_copy(rows_v, acc_sh.at[didx_v], add=True)
        return carry

    lax.fori_loop(0, CHUNKS_PER_TILE, step, 0)
    plsc.subcore_barrier()
    pltpu.sync_copy(acc_sh.at[pl.ds(row0, ROWS_PER_TILE), :],
                    out_hbm.at[c, pl.ds(row0, ROWS_PER_TILE), :])


_msg_call = pl.kernel(
    _msg_body,
    out_type=jax.ShapeDtypeStruct((NUM_CORES, N_ACC, D), jnp.float32),
    mesh=_MESH,
    scratch_types=[
        pltpu.VMEM((CHUNK,), jnp.int32),
        pltpu.VMEM((CHUNK,), jnp.int32),
        pltpu.VMEM((CHUNK, D), jnp.float32),
        pltpu.VMEM_SHARED((N_ACC, D), jnp.float32),
        pltpu.SemaphoreType.DMA,
    ],
)


def _dinv_from(d0_ref, d1_ref):
    return lax.rsqrt(d0_ref[...] + d1_ref[...] + 1.0)


def _tc_first_body(x_ref, w_ref, d0_ref, d1_ref, o_ref):
    dinv = _dinv_from(d0_ref, d1_ref)
    o_ref[...] = jnp.dot(x_ref[...], w_ref[...],
                         preferred_element_type=jnp.float32) * dinv


def _tc_mid_body(acc_ref, hp_ref, d0_ref, d1_ref, b_ref, w_ref, o_ref):
    dinv = _dinv_from(d0_ref, d1_ref)
    agg = (acc_ref[0, :N, :] + acc_ref[1, :N, :] + hp_ref[...]) * dinv
    h = jnp.maximum(agg + b_ref[...], 0.0)
    o_ref[...] = jnp.dot(h, w_ref[...],
                         preferred_element_type=jnp.float32) * dinv


def _tc_head_body(acc_ref, hp_ref, d0_ref, d1_ref, b3_ref, batch_ref,
                  l1w_ref, l1b_ref, l2w_ref, l2b_ref, o_ref):
    dinv = _dinv_from(d0_ref, d1_ref)
    h3 = (acc_ref[0, :N, :] + acc_ref[1, :N, :] + hp_ref[...]) * dinv + b3_ref[...]
    gids = lax.broadcasted_iota(jnp.int32, (G, N), 0)
    onehot = (gids == batch_ref[...]).astype(jnp.float32)
    counts = jnp.sum(onehot, axis=1, keepdims=True)
    pooled = jnp.dot(onehot, h3, preferred_element_type=jnp.float32)
    pooled = pooled / jnp.maximum(counts, 1.0)
    z = jnp.maximum(jnp.dot(pooled, l1w_ref[...],
                            preferred_element_type=jnp.float32) + l1b_ref[...], 0.0)
    o_ref[...] = jnp.dot(z, l2w_ref[...],
                         preferred_element_type=jnp.float32) + l2b_ref[...]


_tc_first = pl.pallas_call(
    _tc_first_body, out_shape=jax.ShapeDtypeStruct((N, D), jnp.float32))
_tc_mid = pl.pallas_call(
    _tc_mid_body, out_shape=jax.ShapeDtypeStruct((N, D), jnp.float32))
_tc_head = pl.pallas_call(
    _tc_head_body, out_shape=jax.ShapeDtypeStruct((G, OUT), jnp.float32))


@jax.jit
def kernel(x, edge_index, batch, W1, b1, W2, b2, W3, b3,
           lin1_W, lin1_b, lin2_W, lin2_b):
    pad = E_PAD - E
    srcp = jnp.concatenate([edge_index[0], jnp.zeros((pad,), jnp.int32)])
    dstp = jnp.concatenate([edge_index[1], jnp.full((pad,), DUMMY, jnp.int32)])
    zeros128 = jnp.zeros((N_ACC, D), jnp.float32)
    deg_flat = _deg_call(dstp, jnp.zeros((HIST,), jnp.float32))
    d0 = deg_flat[:N].reshape(N, 1)
    d1 = deg_flat[HIST:HIST + N].reshape(N, 1)
    h1p = _tc_first(x, W1, d0, d1)
    acc1 = _msg_call(h1p, srcp, dstp, zeros128)
    h2p = _tc_mid(acc1, h1p, d0, d1, b1.reshape(1, D), W2)
    acc2 = _msg_call(h2p, srcp, dstp, zeros128)
    h3p = _tc_mid(acc2, h2p, d0, d1, b2.reshape(1, D), W3)
    acc3 = _msg_call(h3p, srcp, dstp, zeros128)
    return _tc_head(acc3, h3p, d0, d1, b3.reshape(1, D), batch.reshape(1, N),
                    lin1_W, lin1_b.reshape(1, D), lin2_W,
                    lin2_b.reshape(1, OUT))
